# baseline (speedup 1.0000x reference)
import functools

import jax
import jax.numpy as jnp
import numpy as np
from jax import lax
from jax.experimental import pallas as pl
from jax.experimental.pallas import tpu as pltpu
from jax.experimental.pallas import tpu_sc as plsc

N_NODES = 10000
EMB = 128
N_EDGES = 320000
N_LAYERS = 2

NW = 32
CHUNK = 128
EPW = 80
BLK = 16
EPW_C0 = 48
E_PAD = NW * EPW * CHUNK
AGG_ROWS = 10240
CTAB = 512


NODE_PAD = 12288
ATOM_ROWS = 9 * EMB


def _sc_atom_embed(atab_flat, aidx_flat):
    mesh = plsc.VectorSubcoreMesh(core_axis_name="c", subcore_axis_name="s")

    @functools.partial(
        pl.kernel,
        out_type=jax.ShapeDtypeStruct((NODE_PAD, EMB), jnp.float32),
        mesh=mesh,
        scratch_types=[
            pltpu.VMEM((27 * CHUNK,), jnp.int32),
            pltpu.VMEM((CHUNK, EMB), jnp.float32),
            pltpu.VMEM_SHARED((ATOM_ROWS, EMB), jnp.float32),
            pltpu.SemaphoreType.DMA,
        ],
    )
    def k(atab_hbm, aidx_hbm, out_hbm, idx_v, buf, atab_sp, sem):
        c = lax.axis_index("c")
        s = lax.axis_index("s")
        wid = c * 16 + s

        @pl.when(s == 0)
        def _():
            pltpu.sync_copy(atab_hbm, atab_sp)

        plsc.subcore_barrier()
        pltpu.sync_copy(aidx_hbm.at[pl.ds(wid * 27 * CHUNK, 27 * CHUNK)],
                        idx_v)
        for jj in range(3):
            for i in range(9):
                pltpu.async_copy(
                    atab_sp.at[idx_v.at[pl.ds((jj * 9 + i) * CHUNK, CHUNK)]],
                    buf, sem, add=(i > 0)).wait()
            pltpu.sync_copy(
                buf, out_hbm.at[pl.ds((wid * 3 + jj) * CHUNK, CHUNK)])

    return k(atab_flat, aidx_flat)


def _sc_aggregate(h, ctab_l, src2d, dst2d, code2d):
    mesh = plsc.VectorSubcoreMesh(core_axis_name="c", subcore_axis_name="s")

    @functools.partial(
        pl.kernel,
        out_type=jax.ShapeDtypeStruct((2, N_NODES, EMB), jnp.float32),
        mesh=mesh,
        scratch_types=[
            pltpu.VMEM((BLK, CHUNK), jnp.int32),
            pltpu.VMEM((BLK, CHUNK), jnp.int32),
            pltpu.VMEM((BLK, CHUNK), jnp.int32),
            pltpu.VMEM((CHUNK, EMB), jnp.float32),
            pltpu.VMEM((CHUNK, EMB), jnp.float32),
            pltpu.VMEM_SHARED((AGG_ROWS, EMB), jnp.float32),
            pltpu.VMEM_SHARED((CTAB, EMB), jnp.float32),
            pltpu.SemaphoreType.DMA,
            pltpu.SemaphoreType.DMA,
            pltpu.SemaphoreType.DMA,
            pltpu.SemaphoreType.DMA,
            pltpu.SemaphoreType.DMA,
        ],
    )
    def k(h_hbm, ctab_hbm, src_hbm, dst_hbm, code_hbm, out_hbm,
          src_v, dst_v, code_v, buf0, buf1, aggr_sp, ctab_sp,
          sem_g0, sem_g1, sem_s0, sem_s1, sem_a):
        bufs = (buf0, buf1)
        sem_g = (sem_g0, sem_g1)
        sem_s = (sem_s0, sem_s1)
        c = lax.axis_index("c")
        s = lax.axis_index("s")
        wid = c * 16 + s

        def zrow(i, carry):
            for g in range(8):
                buf0[i, pl.ds(g * 16, 16)] = jnp.zeros((16,), jnp.float32)
            return carry
        lax.fori_loop(0, 128, zrow, 0)
        zbase = pl.multiple_of(s * 640, 8)
        for t in range(5):
            pltpu.sync_copy(buf0, aggr_sp.at[pl.ds(zbase + t * 128, 128)])

        @pl.when(s == 0)
        def _():
            pltpu.sync_copy(ctab_hbm, ctab_sp)

        plsc.subcore_barrier()

        def relu_buf(buf):
            def rrow(i, c2):
                for g in range(8):
                    sl = pl.ds(g * 16, 16)
                    buf[i, sl] = jnp.maximum(buf[i, sl], 0.0)
                return c2
            lax.fori_loop(0, CHUNK, rrow, 0)

        nblk = lax.select(c == 0, EPW_C0 // BLK, (160 - EPW_C0) // BLK)
        cbase = s * 160 + c * EPW_C0

        def block(blk, carry):
            base = cbase + blk * BLK
            pltpu.sync_copy(src_hbm.at[pl.ds(base, BLK)], src_v)
            pltpu.sync_copy(dst_hbm.at[pl.ds(base, BLK)], dst_v)
            pltpu.sync_copy(code_hbm.at[pl.ds(base, BLK)], code_v)
            g_desc = [None, None]
            s_desc = [None, None]
            g_desc[0] = pltpu.async_copy(h_hbm.at[src_v.at[0]], bufs[0],
                                         sem_g[0])
            for t in range(BLK):
                p = t & 1
                buf = bufs[p]
                g_desc[p].wait()
                pltpu.async_copy(ctab_sp.at[code_v.at[t]], buf, sem_a,
                                 add=True).wait()
                if t + 1 < BLK:
                    if t >= 1:
                        s_desc[p ^ 1].wait()
                    g_desc[p ^ 1] = pltpu.async_copy(
                        h_hbm.at[src_v.at[t + 1]], bufs[p ^ 1], sem_g[p ^ 1])
                relu_buf(buf)
                s_desc[p] = pltpu.async_copy(buf, aggr_sp.at[dst_v.at[t]],
                                             sem_s[p], add=True)
            s_desc[0].wait()
            s_desc[1].wait()
            return carry
        lax.fori_loop(0, nblk, block, 0)

        plsc.subcore_barrier()

        dbase = pl.multiple_of(s * 640, 8)

        @pl.when(s < 15)
        def _():
            pltpu.sync_copy(aggr_sp.at[pl.ds(dbase, 640)],
                            out_hbm.at[c, pl.ds(dbase, 640)])

        @pl.when(s == 15)
        def _():
            pltpu.sync_copy(aggr_sp.at[pl.ds(9600, 400)],
                            out_hbm.at[c, pl.ds(9600, 400)])

    return k(h, ctab_l, src2d, dst2d, code2d)


def _dense_mm1(h, parts, eps, w1, b1):
    def body(h_ref, parts_ref, eps_ref, w1_ref, b1_ref, a_ref):
        u = ((1.0 + eps_ref[0, 0]) * h_ref[...]
             + parts_ref[pl.ds(0, N_NODES), :]
             + parts_ref[pl.ds(N_NODES, N_NODES), :])
        a_ref[...] = jnp.dot(u, w1_ref[...],
                             preferred_element_type=jnp.float32) + b1_ref[...]

    return pl.pallas_call(
        body,
        out_shape=jax.ShapeDtypeStruct((N_NODES, 2 * EMB), jnp.float32),
    )(h, parts, eps, w1, b1)


def _dense_mm2(a, m1, v1, g1, bb1, w2, b2):
    def body(a_ref, m_ref, v_ref, g_ref, bb_ref, w2_ref, b2_ref, z_ref):
        an = (g_ref[...] * (a_ref[...] - m_ref[...])
              / jnp.sqrt(v_ref[...] + 1e-5) + bb_ref[...])
        an = jnp.maximum(an, 0.0)
        z_ref[...] = jnp.dot(an, w2_ref[...],
                             preferred_element_type=jnp.float32) + b2_ref[...]

    return pl.pallas_call(
        body,
        out_shape=jax.ShapeDtypeStruct((N_NODES, EMB), jnp.float32),
    )(a, m1, v1, g1, bb1, w2, b2)


def _dense_bn2(z, m2, v2, g2, bb2, final_relu):
    def body(z_ref, m_ref, v_ref, g_ref, bb_ref, o_ref):
        o = (g_ref[...] * (z_ref[...] - m_ref[...])
             / jnp.sqrt(v_ref[...] + 1e-5) + bb_ref[...])
        if final_relu:
            o = jnp.maximum(o, 0.0)
        o_ref[...] = o

    return pl.pallas_call(
        body,
        out_shape=jax.ShapeDtypeStruct((N_NODES, EMB), jnp.float32),
    )(z, m2, v2, g2, bb2)


def _dense(h, parts, eps, w1, b1, g1, bb1, w2, b2, g2, bb2, final_relu):
    a = _dense_mm1(h, parts, eps, w1, b1)
    u = (1.0 + eps[0, 0]) * h + parts[:N_NODES] + parts[N_NODES:]
    a_x = u @ w1 + b1[0]
    m1 = jnp.mean(a_x, axis=0, keepdims=True)
    v1 = jnp.var(a_x, axis=0, keepdims=True)
    z = _dense_mm2(a, m1, v1, g1, bb1, w2, b2)
    an_x = jnp.maximum(g1[0] * (a_x - m1[0]) / jnp.sqrt(v1[0] + 1e-5) + bb1[0],
                       0.0)
    z_x = an_x @ w2 + b2[0]
    m2 = jnp.mean(z_x, axis=0, keepdims=True)
    v2 = jnp.var(z_x, axis=0, keepdims=True)
    return _dense_bn2(z, m2, v2, g2, bb2, final_relu)


def kernel(x, edge_index, edge_attr, batch, params):
    del batch
    aemb_flat = params['atom_emb'].reshape(ATOM_ROWS, EMB)
    bflat = jnp.concatenate(
        [params['bond_emb_%d' % l].reshape(24, EMB) for l in range(N_LAYERS)],
        axis=0)
    cs = jnp.arange(N_LAYERS * CTAB, dtype=jnp.int32)
    lidx = cs // CTAB
    cc = cs % CTAB
    ctab = (bflat[lidx * 24 + cc // 64]
            + bflat[lidx * 24 + 8 + (cc // 8) % 8]
            + bflat[lidx * 24 + 16 + cc % 8])

    xt = x.T.astype(jnp.int32) + (jnp.arange(9, dtype=jnp.int32) * EMB)[:, None]
    xt = jnp.pad(xt, ((0, 0), (0, NODE_PAD - N_NODES)))
    aidx = jnp.transpose(xt.reshape(9, NODE_PAD // CHUNK, CHUNK),
                         (1, 0, 2)).reshape(-1)
    h = _sc_atom_embed(aemb_flat, aidx)[:N_NODES]

    src = edge_index[0]
    dst = edge_index[1]
    code = (edge_attr[:, 0] * 64 + edge_attr[:, 1] * 8 + edge_attr[:, 2]
            ).astype(jnp.int32)
    pad = E_PAD - N_EDGES
    src_p = jnp.concatenate([src, jnp.zeros((pad,), jnp.int32)])
    dst_p = jnp.concatenate(
        [dst, N_NODES + (jnp.arange(pad, dtype=jnp.int32) % 16)])
    code_p = jnp.concatenate([code, jnp.zeros((pad,), jnp.int32)])
    src2d = src_p.reshape(NW * EPW, CHUNK)
    dst2d = dst_p.reshape(NW * EPW, CHUNK)
    code2d = code_p.reshape(NW * EPW, CHUNK)

    for l in range(N_LAYERS):
        parts = _sc_aggregate(h, ctab[l * CTAB:(l + 1) * CTAB], src2d, dst2d,
                              code2d)
        h = _dense(
            h, parts.reshape(2 * N_NODES, EMB),
            params['eps_%d' % l].reshape(1, 1),
            params['W1_%d' % l], params['b1_%d' % l].reshape(1, 2 * EMB),
            params['bn1_g_%d' % l].reshape(1, 2 * EMB),
            params['bn1_b_%d' % l].reshape(1, 2 * EMB),
            params['W2_%d' % l], params['b2_%d' % l].reshape(1, EMB),
            params['bn_g_%d' % l].reshape(1, EMB),
            params['bn_b_%d' % l].reshape(1, EMB),
            final_relu=(l < N_LAYERS - 1))
    return h


# speedup vs baseline: 1.3929x; 1.3929x over previous
import functools

import jax
import jax.numpy as jnp
import numpy as np
from jax import lax
from jax.experimental import pallas as pl
from jax.experimental.pallas import tpu as pltpu
from jax.experimental.pallas import tpu_sc as plsc

N_NODES = 10000
EMB = 128
N_EDGES = 320000
N_LAYERS = 2

NW = 32
CHUNK = 128
EPW = 80
BLK = 16
EPW_C0 = 112
E_PAD = NW * EPW * CHUNK
AGG_ROWS = 10240
CTAB = 512


NODE_PAD = 12288
ATOM_ROWS = 9 * EMB


def _sc_atom_embed(atab_flat, aidx_flat):
    mesh = plsc.VectorSubcoreMesh(core_axis_name="c", subcore_axis_name="s")

    @functools.partial(
        pl.kernel,
        out_type=jax.ShapeDtypeStruct((NODE_PAD, EMB), jnp.float32),
        mesh=mesh,
        scratch_types=[
            pltpu.VMEM((27 * CHUNK,), jnp.int32),
            pltpu.VMEM((CHUNK, EMB), jnp.float32),
            pltpu.VMEM_SHARED((ATOM_ROWS, EMB), jnp.float32),
            pltpu.SemaphoreType.DMA,
        ],
    )
    def k(atab_hbm, aidx_hbm, out_hbm, idx_v, buf, atab_sp, sem):
        c = lax.axis_index("c")
        s = lax.axis_index("s")
        wid = c * 16 + s

        @pl.when(s == 0)
        def _():
            pltpu.sync_copy(atab_hbm, atab_sp)

        plsc.subcore_barrier()
        pltpu.sync_copy(aidx_hbm.at[pl.ds(wid * 27 * CHUNK, 27 * CHUNK)],
                        idx_v)
        for jj in range(3):
            for i in range(9):
                pltpu.async_copy(
                    atab_sp.at[idx_v.at[pl.ds((jj * 9 + i) * CHUNK, CHUNK)]],
                    buf, sem, add=(i > 0)).wait()
            pltpu.sync_copy(
                buf, out_hbm.at[pl.ds((wid * 3 + jj) * CHUNK, CHUNK)])

    return k(atab_flat, aidx_flat)


def _sc_aggregate(h, ctab_l, src2d, dst2d, code2d):
    mesh = plsc.VectorSubcoreMesh(core_axis_name="c", subcore_axis_name="s")

    @functools.partial(
        pl.kernel,
        out_type=jax.ShapeDtypeStruct((2, N_NODES, EMB), jnp.float32),
        mesh=mesh,
        scratch_types=[
            pltpu.VMEM((BLK, CHUNK), jnp.int32),
            pltpu.VMEM((BLK, CHUNK), jnp.int32),
            pltpu.VMEM((BLK, CHUNK), jnp.int32),
            pltpu.VMEM((CHUNK, EMB), jnp.float32),
            pltpu.VMEM((CHUNK, EMB), jnp.float32),
            pltpu.VMEM_SHARED((AGG_ROWS, EMB), jnp.float32),
            pltpu.VMEM_SHARED((CTAB, EMB), jnp.float32),
            pltpu.SemaphoreType.DMA,
            pltpu.SemaphoreType.DMA,
            pltpu.SemaphoreType.DMA,
            pltpu.SemaphoreType.DMA,
            pltpu.SemaphoreType.DMA,
        ],
    )
    def k(h_hbm, ctab_hbm, src_hbm, dst_hbm, code_hbm, out_hbm,
          src_v, dst_v, code_v, buf0, buf1, aggr_sp, ctab_sp,
          sem_g0, sem_g1, sem_s0, sem_s1, sem_a):
        bufs = (buf0, buf1)
        sem_g = (sem_g0, sem_g1)
        sem_s = (sem_s0, sem_s1)
        c = lax.axis_index("c")
        s = lax.axis_index("s")
        wid = c * 16 + s

        def zrow(i, carry):
            for g in range(8):
                buf0[i, pl.ds(g * 16, 16)] = jnp.zeros((16,), jnp.float32)
            return carry
        lax.fori_loop(0, 128, zrow, 0)
        zbase = pl.multiple_of(s * 640, 8)
        for t in range(5):
            pltpu.sync_copy(buf0, aggr_sp.at[pl.ds(zbase + t * 128, 128)])

        @pl.when(s == 0)
        def _():
            pltpu.sync_copy(ctab_hbm, ctab_sp)

        plsc.subcore_barrier()

        def relu_buf(buf):
            def rrow(i, c2):
                for g in range(8):
                    sl = pl.ds(g * 16, 16)
                    buf[i, sl] = jnp.maximum(buf[i, sl], 0.0)
                return c2
            lax.fori_loop(0, CHUNK, rrow, 0)

        nblk = lax.select(c == 0, EPW_C0 // BLK, (160 - EPW_C0) // BLK)
        cbase = s * 160 + c * EPW_C0

        def block(blk, carry):
            base = cbase + blk * BLK
            pltpu.sync_copy(src_hbm.at[pl.ds(base, BLK)], src_v)
            pltpu.sync_copy(dst_hbm.at[pl.ds(base, BLK)], dst_v)
            pltpu.sync_copy(code_hbm.at[pl.ds(base, BLK)], code_v)
            g_desc = [None, None]
            s_desc = [None, None]
            g_desc[0] = pltpu.async_copy(h_hbm.at[src_v.at[0]], bufs[0],
                                         sem_g[0])
            for t in range(BLK):
                p = t & 1
                buf = bufs[p]
                g_desc[p].wait()
                pltpu.async_copy(ctab_sp.at[code_v.at[t]], buf, sem_a,
                                 add=True).wait()
                if t + 1 < BLK:
                    if t >= 1:
                        s_desc[p ^ 1].wait()
                    g_desc[p ^ 1] = pltpu.async_copy(
                        h_hbm.at[src_v.at[t + 1]], bufs[p ^ 1], sem_g[p ^ 1])
                relu_buf(buf)
                s_desc[p] = pltpu.async_copy(buf, aggr_sp.at[dst_v.at[t]],
                                             sem_s[p], add=True)
            s_desc[0].wait()
            s_desc[1].wait()
            return carry
        lax.fori_loop(0, nblk, block, 0)

        plsc.subcore_barrier()

        dbase = pl.multiple_of(s * 640, 8)

        @pl.when(s < 15)
        def _():
            pltpu.sync_copy(aggr_sp.at[pl.ds(dbase, 640)],
                            out_hbm.at[c, pl.ds(dbase, 640)])

        @pl.when(s == 15)
        def _():
            pltpu.sync_copy(aggr_sp.at[pl.ds(9600, 400)],
                            out_hbm.at[c, pl.ds(9600, 400)])

    return k(h, ctab_l, src2d, dst2d, code2d)


def _dense_mm1(h, parts, eps, w1, b1):
    def body(h_ref, parts_ref, eps_ref, w1_ref, b1_ref, a_ref):
        u = ((1.0 + eps_ref[0, 0]) * h_ref[...]
             + parts_ref[pl.ds(0, N_NODES), :]
             + parts_ref[pl.ds(N_NODES, N_NODES), :])
        a_ref[...] = jnp.dot(u, w1_ref[...],
                             preferred_element_type=jnp.float32) + b1_ref[...]

    return pl.pallas_call(
        body,
        out_shape=jax.ShapeDtypeStruct((N_NODES, 2 * EMB), jnp.float32),
    )(h, parts, eps, w1, b1)


def _dense_mm2(a, m1, v1, g1, bb1, w2, b2):
    def body(a_ref, m_ref, v_ref, g_ref, bb_ref, w2_ref, b2_ref, z_ref):
        an = (g_ref[...] * (a_ref[...] - m_ref[...])
              / jnp.sqrt(v_ref[...] + 1e-5) + bb_ref[...])
        an = jnp.maximum(an, 0.0)
        z_ref[...] = jnp.dot(an, w2_ref[...],
                             preferred_element_type=jnp.float32) + b2_ref[...]

    return pl.pallas_call(
        body,
        out_shape=jax.ShapeDtypeStruct((N_NODES, EMB), jnp.float32),
    )(a, m1, v1, g1, bb1, w2, b2)


def _dense_bn2(z, m2, v2, g2, bb2, final_relu):
    def body(z_ref, m_ref, v_ref, g_ref, bb_ref, o_ref):
        o = (g_ref[...] * (z_ref[...] - m_ref[...])
             / jnp.sqrt(v_ref[...] + 1e-5) + bb_ref[...])
        if final_relu:
            o = jnp.maximum(o, 0.0)
        o_ref[...] = o

    return pl.pallas_call(
        body,
        out_shape=jax.ShapeDtypeStruct((N_NODES, EMB), jnp.float32),
    )(z, m2, v2, g2, bb2)


def _dense(h, parts, eps, w1, b1, g1, bb1, w2, b2, g2, bb2, final_relu):
    a = _dense_mm1(h, parts, eps, w1, b1)
    u = (1.0 + eps[0, 0]) * h + parts[:N_NODES] + parts[N_NODES:]
    a_x = u @ w1 + b1[0]
    m1 = jnp.mean(a_x, axis=0, keepdims=True)
    v1 = jnp.var(a_x, axis=0, keepdims=True)
    z = _dense_mm2(a, m1, v1, g1, bb1, w2, b2)
    an_x = jnp.maximum(g1[0] * (a_x - m1[0]) / jnp.sqrt(v1[0] + 1e-5) + bb1[0],
                       0.0)
    z_x = an_x @ w2 + b2[0]
    m2 = jnp.mean(z_x, axis=0, keepdims=True)
    v2 = jnp.var(z_x, axis=0, keepdims=True)
    return _dense_bn2(z, m2, v2, g2, bb2, final_relu)


def kernel(x, edge_index, edge_attr, batch, params):
    del batch
    aemb_flat = params['atom_emb'].reshape(ATOM_ROWS, EMB)
    bflat = jnp.concatenate(
        [params['bond_emb_%d' % l].reshape(24, EMB) for l in range(N_LAYERS)],
        axis=0)
    cs = jnp.arange(N_LAYERS * CTAB, dtype=jnp.int32)
    lidx = cs // CTAB
    cc = cs % CTAB
    ctab = (bflat[lidx * 24 + cc // 64]
            + bflat[lidx * 24 + 8 + (cc // 8) % 8]
            + bflat[lidx * 24 + 16 + cc % 8])

    xt = x.T.astype(jnp.int32) + (jnp.arange(9, dtype=jnp.int32) * EMB)[:, None]
    xt = jnp.pad(xt, ((0, 0), (0, NODE_PAD - N_NODES)))
    aidx = jnp.transpose(xt.reshape(9, NODE_PAD // CHUNK, CHUNK),
                         (1, 0, 2)).reshape(-1)
    h = _sc_atom_embed(aemb_flat, aidx)[:N_NODES]

    src = edge_index[0]
    dst = edge_index[1]
    code = (edge_attr[:, 0] * 64 + edge_attr[:, 1] * 8 + edge_attr[:, 2]
            ).astype(jnp.int32)
    pad = E_PAD - N_EDGES
    src_p = jnp.concatenate([src, jnp.zeros((pad,), jnp.int32)])
    dst_p = jnp.concatenate(
        [dst, N_NODES + (jnp.arange(pad, dtype=jnp.int32) % 16)])
    code_p = jnp.concatenate([code, jnp.zeros((pad,), jnp.int32)])
    src2d = src_p.reshape(NW * EPW, CHUNK)
    dst2d = dst_p.reshape(NW * EPW, CHUNK)
    code2d = code_p.reshape(NW * EPW, CHUNK)

    for l in range(N_LAYERS):
        parts = _sc_aggregate(h, ctab[l * CTAB:(l + 1) * CTAB], src2d, dst2d,
                              code2d)
        h = _dense(
            h, parts.reshape(2 * N_NODES, EMB),
            params['eps_%d' % l].reshape(1, 1),
            params['W1_%d' % l], params['b1_%d' % l].reshape(1, 2 * EMB),
            params['bn1_g_%d' % l].reshape(1, 2 * EMB),
            params['bn1_b_%d' % l].reshape(1, 2 * EMB),
            params['W2_%d' % l], params['b2_%d' % l].reshape(1, EMB),
            params['bn_g_%d' % l].reshape(1, EMB),
            params['bn_b_%d' % l].reshape(1, EMB),
            final_relu=(l < N_LAYERS - 1))
    return h
